# baseline (speedup 1.0000x reference)
import jax
import jax.numpy as jnp
from jax.experimental import pallas as pl
from jax.experimental.pallas import tpu as pltpu

BATCH = 128
R = 64
NUM_ADAPTERS = 64
D_OUT = 4096


def _matmul_body(wids_ref, y_ref, b_ref, out_ref, acc_ref):
    a = pl.program_id(0)

    @pl.when(a == 0)
    def _():
        acc_ref[...] = jnp.zeros_like(acc_ref)

    mask = wids_ref[...] == a
    y_masked = jnp.where(mask, y_ref[...], jnp.zeros_like(y_ref))
    w = pltpu.bitcast(b_ref[0], jnp.uint32)
    mag = ((w >> 3) & jnp.uint32(0x0FFF0FFF)) + jnp.uint32(0x38003800)
    bf = (w & jnp.uint32(0x80008000)) | mag
    b_bf16 = pltpu.bitcast(bf, jnp.bfloat16)
    acc_ref[...] += jnp.dot(y_masked.astype(jnp.bfloat16), b_bf16,
                            preferred_element_type=jnp.float32)

    @pl.when(a == NUM_ADAPTERS - 1)
    def _():
        out_ref[...] = (acc_ref[...] * 2.0).astype(out_ref.dtype)


def kernel(y, wids, lora_B):
    y2d = y.reshape(BATCH, R).astype(jnp.float32)
    wids2d = wids.reshape(BATCH, 1)

    out = pl.pallas_call(
        _matmul_body,
        grid=(NUM_ADAPTERS,),
        in_specs=[
            pl.BlockSpec((BATCH, 1), lambda a: (0, 0)),
            pl.BlockSpec((BATCH, R), lambda a: (0, 0)),
            pl.BlockSpec((1, R, D_OUT), lambda a: (a, 0, 0)),
        ],
        out_specs=pl.BlockSpec((BATCH, D_OUT), lambda a: (0, 0)),
        out_shape=jax.ShapeDtypeStruct((BATCH, D_OUT), jnp.float32),
        scratch_shapes=[pltpu.VMEM((BATCH, D_OUT), jnp.float32)],
        compiler_params=pltpu.CompilerParams(
            dimension_semantics=("arbitrary",),
        ),
    )(wids2d, y2d, jax.lax.bitcast_convert_type(lora_B, jnp.uint16))
    return out.astype(y.dtype).reshape(BATCH, 1, D_OUT)


# speedup vs baseline: 1.6003x; 1.6003x over previous
import jax
import jax.numpy as jnp
from jax.experimental import pallas as pl
from jax.experimental.pallas import tpu as pltpu

BATCH = 128
R = 64
NUM_ADAPTERS = 64
D_OUT = 4096
BLK = 8


def _f16_bits_to_bf16(bits_2d):
    w = pltpu.bitcast(bits_2d, jnp.uint32)
    mag = ((w >> 3) & jnp.uint32(0x0FFF0FFF)) + jnp.uint32(0x38003800)
    bf = (w & jnp.uint32(0x80008000)) | mag
    return pltpu.bitcast(bf, jnp.bfloat16)


def _matmul_body(wids_ref, y_ref, b_ref, out_ref, acc_ref):
    a = pl.program_id(0)

    @pl.when(a == 0)
    def _():
        acc_ref[...] = jnp.zeros_like(acc_ref)

    y = y_ref[...]
    wids = wids_ref[...]
    acc = acc_ref[...]
    for j in range(BLK):
        ad = a * BLK + j
        mask = wids == ad
        y_masked = jnp.where(mask, y, jnp.zeros_like(y))
        b_bf16 = _f16_bits_to_bf16(b_ref[j])
        acc += jnp.dot(y_masked.astype(jnp.bfloat16), b_bf16,
                       preferred_element_type=jnp.float32)
    acc_ref[...] = acc

    @pl.when(a == NUM_ADAPTERS // BLK - 1)
    def _():
        out_ref[...] = (acc_ref[...] * 2.0).astype(out_ref.dtype)


def kernel(y, wids, lora_B):
    y2d = y.reshape(BATCH, R).astype(jnp.float32)
    wids2d = wids.reshape(BATCH, 1)

    out = pl.pallas_call(
        _matmul_body,
        grid=(NUM_ADAPTERS // BLK,),
        in_specs=[
            pl.BlockSpec((BATCH, 1), lambda a: (0, 0)),
            pl.BlockSpec((BATCH, R), lambda a: (0, 0)),
            pl.BlockSpec((BLK, R, D_OUT), lambda a: (a, 0, 0)),
        ],
        out_specs=pl.BlockSpec((BATCH, D_OUT), lambda a: (0, 0)),
        out_shape=jax.ShapeDtypeStruct((BATCH, D_OUT), jnp.float32),
        scratch_shapes=[pltpu.VMEM((BATCH, D_OUT), jnp.float32)],
        compiler_params=pltpu.CompilerParams(
            dimension_semantics=("arbitrary",),
        ),
    )(wids2d, y2d, jax.lax.bitcast_convert_type(lora_B, jnp.bfloat16))
    return out.astype(y.dtype).reshape(BATCH, 1, D_OUT)


# speedup vs baseline: 1.7885x; 1.1176x over previous
import jax
import jax.numpy as jnp
from jax.experimental import pallas as pl
from jax.experimental.pallas import tpu as pltpu

BATCH = 128
R = 64
NUM_ADAPTERS = 64
D_OUT = 4096
BLK = 8
GRID = NUM_ADAPTERS // BLK


def _matmul_body(wids_ref, y_ref, b_ref, out_ref, acc_ref, yd_ref):
    a = pl.program_id(0)

    @pl.when(a == 0)
    def _():
        acc_ref[...] = jnp.zeros_like(acc_ref)
        y = y_ref[...]
        wids = wids_ref[...]
        for ad in range(NUM_ADAPTERS):
            mask = wids == ad
            y_m = jnp.where(mask, y, jnp.zeros_like(y))
            yd_ref[:, ad * R:(ad + 1) * R] = y_m.astype(jnp.bfloat16)

    yd = yd_ref[:, pl.ds(a * (BLK * R), BLK * R)]
    b = b_ref[...].reshape(BLK * R, D_OUT)
    acc_ref[...] += jnp.dot(yd, b, preferred_element_type=jnp.float32)

    @pl.when(a == GRID - 1)
    def _():
        out_ref[...] = (acc_ref[...] * 2.0).astype(out_ref.dtype)


def kernel(y, wids, lora_B):
    y2d = y.reshape(BATCH, R).astype(jnp.float32)
    wids2d = wids.reshape(BATCH, 1)

    out = pl.pallas_call(
        _matmul_body,
        grid=(GRID,),
        in_specs=[
            pl.BlockSpec((BATCH, 1), lambda a: (0, 0)),
            pl.BlockSpec((BATCH, R), lambda a: (0, 0)),
            pl.BlockSpec((BLK, R, D_OUT), lambda a: (a, 0, 0)),
        ],
        out_specs=pl.BlockSpec((BATCH, D_OUT), lambda a: (0, 0)),
        out_shape=jax.ShapeDtypeStruct((BATCH, D_OUT), jnp.float32),
        scratch_shapes=[
            pltpu.VMEM((BATCH, D_OUT), jnp.float32),
            pltpu.VMEM((BATCH, NUM_ADAPTERS * R), jnp.bfloat16),
        ],
        compiler_params=pltpu.CompilerParams(
            dimension_semantics=("arbitrary",),
        ),
    )(wids2d, y2d, lora_B.astype(jnp.bfloat16))
    return out.astype(y.dtype).reshape(BATCH, 1, D_OUT)


# speedup vs baseline: 1.8399x; 1.0287x over previous
import jax
import jax.numpy as jnp
from jax.experimental import pallas as pl
from jax.experimental.pallas import tpu as pltpu

BATCH = 128
R = 64
NUM_ADAPTERS = 64
D_OUT = 4096
BLK = 16
GRID = NUM_ADAPTERS // BLK


def _matmul_body(wids_ref, y_ref, b_ref, out_ref, acc_ref, yd_ref):
    a = pl.program_id(0)

    @pl.when(a == 0)
    def _():
        acc_ref[...] = jnp.zeros_like(acc_ref)
        y = y_ref[...]
        wids = wids_ref[...]
        for ad in range(NUM_ADAPTERS):
            mask = wids == ad
            y_m = jnp.where(mask, y, jnp.zeros_like(y))
            yd_ref[:, ad * R:(ad + 1) * R] = y_m.astype(jnp.bfloat16)

    yd = yd_ref[:, pl.ds(a * (BLK * R), BLK * R)]
    b = b_ref[...].reshape(BLK * R, D_OUT)
    acc_ref[...] += jnp.dot(yd, b, preferred_element_type=jnp.float32)

    @pl.when(a == GRID - 1)
    def _():
        out_ref[...] = (acc_ref[...] * 2.0).astype(out_ref.dtype)


def kernel(y, wids, lora_B):
    y2d = y.reshape(BATCH, R).astype(jnp.float32)
    wids2d = wids.reshape(BATCH, 1)

    out = pl.pallas_call(
        _matmul_body,
        grid=(GRID,),
        in_specs=[
            pl.BlockSpec((BATCH, 1), lambda a: (0, 0)),
            pl.BlockSpec((BATCH, R), lambda a: (0, 0)),
            pl.BlockSpec((BLK, R, D_OUT), lambda a: (a, 0, 0)),
        ],
        out_specs=pl.BlockSpec((BATCH, D_OUT), lambda a: (0, 0)),
        out_shape=jax.ShapeDtypeStruct((BATCH, D_OUT), jnp.float32),
        scratch_shapes=[
            pltpu.VMEM((BATCH, D_OUT), jnp.float32),
            pltpu.VMEM((BATCH, NUM_ADAPTERS * R), jnp.bfloat16),
        ],
        compiler_params=pltpu.CompilerParams(
            dimension_semantics=("arbitrary",),
        ),
    )(wids2d, y2d, lora_B.astype(jnp.bfloat16))
    return out.astype(y.dtype).reshape(BATCH, 1, D_OUT)
